# Initial kernel scaffold; baseline (speedup 1.0000x reference)
import jax
import jax.numpy as jnp
from jax.experimental import pallas as pl


def kernel(tokens, vocab_map):
    raise NotImplementedError("write your pallas kernel here")


# speedup vs baseline: 288.7635x; 288.7635x over previous
import functools

import jax
import jax.numpy as jnp
from jax import lax
from jax.experimental import pallas as pl
from jax.experimental.pallas import tpu as pltpu
from jax.experimental.pallas import tpu_sc as plsc

_VOCAB = 100000
_N_TOKENS = 3276800

_NC = 2
_NS = 16
_L = 16
_NW = _NC * _NS
_B_PER_W = _N_TOKENS // _NW
_CHUNK = 5120
_N_CHUNKS = _B_PER_W // _CHUNK


def _lookup_kernel(tok_hbm, map_hbm, out_hbm, table_v, idx_v, res_v):
    wid = lax.axis_index("s") * _NC + lax.axis_index("c")
    base = wid * _B_PER_W

    pltpu.sync_copy(map_hbm, table_v)

    def chunk_body(ci, _):
        off = base + ci * _CHUNK
        pltpu.sync_copy(tok_hbm.at[pl.ds(off, _CHUNK)], idx_v)

        def gather_body(i, _):
            ids = idx_v[pl.ds(i * _L, _L)]
            res_v[pl.ds(i * _L, _L)] = plsc.load_gather(table_v, [ids])
            return 0

        lax.fori_loop(0, _CHUNK // _L, gather_body, 0, unroll=8)
        pltpu.sync_copy(res_v, out_hbm.at[pl.ds(off, _CHUNK)])
        return 0

    lax.fori_loop(0, _N_CHUNKS, chunk_body, 0)


def kernel(tokens, vocab_map):
    mesh = plsc.VectorSubcoreMesh(core_axis_name="c", subcore_axis_name="s")
    run = functools.partial(
        pl.kernel,
        mesh=mesh,
        out_type=jax.ShapeDtypeStruct((_N_TOKENS,), jnp.int32),
        scratch_types=[
            pltpu.VMEM((_VOCAB,), jnp.int32),
            pltpu.VMEM((_CHUNK,), jnp.int32),
            pltpu.VMEM((_CHUNK,), jnp.int32),
        ],
        compiler_params=pltpu.CompilerParams(needs_layout_passes=False),
    )(_lookup_kernel)
    return run(tokens, vocab_map)
